# Initial kernel scaffold; baseline (speedup 1.0000x reference)
import jax
import jax.numpy as jnp
from jax.experimental import pallas as pl


def kernel(H, input_ids, emb, depth_emb, ln_g, ln_b, W1, b1, W2, b2, Wq, bq, Wb, bb):
    raise NotImplementedError("write your pallas kernel here")


# speedup vs baseline: 1.4350x; 1.4350x over previous
import functools
import math

import jax
import jax.numpy as jnp
from jax.experimental import pallas as pl
from jax.experimental.pallas import tpu as pltpu
from jax.experimental.pallas import tpu_sc as plsc

B, T, D, V, DM, DH = 1, 2048, 4, 8192, 1024, 4096
LN_EPS = 1e-05

_GATHER_W = 128
_GK = 4
_GC = DM // _GK
_TT = 64
_RB = 256


def _sc_gather(emb4, ids_exp):
    mesh = plsc.VectorSubcoreMesh(core_axis_name="core", subcore_axis_name="subcore")
    n_idx = T * _GK

    @functools.partial(
        pl.kernel,
        out_type=jax.ShapeDtypeStruct((n_idx, _GC), emb4.dtype),
        mesh=mesh,
    )
    def gather_kernel(emb_hbm, ids_hbm, out_hbm):
        def body(i_vmem, o_vmem):
            pltpu.sync_copy(emb_hbm.at[i_vmem.at[0]], o_vmem)

        pltpu.emit_pipeline(
            body,
            grid=(n_idx // _GATHER_W,),
            in_specs=[pl.BlockSpec((1, _GATHER_W), index_map=lambda i: (0, i))],
            out_specs=[pl.BlockSpec((_GATHER_W, _GC), index_map=lambda i: (i, 0))],
            core_axis_name=("core", "subcore"),
            dimension_semantics=(pltpu.PARALLEL,),
        )(ids_hbm, out_hbm)

    return gather_kernel(emb4, ids_exp).reshape(T, DM)


def _mlp_body(h_ref, e_ref, demb_ref, lng_ref, lnb_ref, w1_ref, b1_ref,
              w2_ref, b2_ref, wb_ref, bb_ref, s2_ref, beta_ref):
    R = _TT * D
    x = h_ref[...] + e_ref[...]
    x4 = jnp.broadcast_to(x[:, None, :], (_TT, D, DM)).reshape(R, DM)
    d4 = jnp.broadcast_to(demb_ref[...][None, :, :], (_TT, D, DM)).reshape(R, DM)
    s = x4 + d4
    mu = jnp.mean(s, axis=1, keepdims=True)
    c = s - mu
    var = jnp.mean(c * c, axis=1, keepdims=True)
    sn = c * jax.lax.rsqrt(var + LN_EPS) * lng_ref[...] + lnb_ref[...]
    hpre = jnp.dot(sn.astype(jnp.bfloat16), w1_ref[...],
                   preferred_element_type=jnp.float32) + b1_ref[...]
    h = 0.5 * hpre * (1.0 + jax.lax.erf(hpre * (1.0 / math.sqrt(2.0))))
    s2 = jnp.dot(h.astype(jnp.bfloat16), w2_ref[...],
                 preferred_element_type=jnp.float32) + b2_ref[...]
    s2_ref[...] = s2.astype(jnp.bfloat16)
    beta_ref[...] = (jnp.sum(s2 * wb_ref[...], axis=1) + bb_ref[0, 0]).reshape(1, 1, R)


def _q_body(s2_ref, wq_ref, bq_ref, q_ref):
    q_ref[...] = jnp.dot(s2_ref[...], wq_ref[...],
                         preferred_element_type=jnp.float32) + bq_ref[...]


def kernel(H, input_ids, emb, depth_emb, ln_g, ln_b, W1, b1, W2, b2, Wq, bq, Wb, bb):
    f32 = jnp.float32
    bf16 = jnp.bfloat16

    H2 = H.reshape(T, DM)
    ids = input_ids.reshape(T).astype(jnp.int32)
    ids_exp = (ids[:, None] * _GK + jnp.arange(_GK, dtype=jnp.int32)[None, :]
               ).reshape(1, T * _GK)

    E = _sc_gather(emb.reshape(V * _GK, _GC), ids_exp)

    R = _TT * D
    n_a = T // _TT
    s2_flat, beta2 = pl.pallas_call(
        _mlp_body,
        grid=(n_a,),
        in_specs=[
            pl.BlockSpec((_TT, DM), lambda i: (i, 0)),
            pl.BlockSpec((_TT, DM), lambda i: (i, 0)),
            pl.BlockSpec((D, DM), lambda i: (0, 0)),
            pl.BlockSpec((1, DM), lambda i: (0, 0)),
            pl.BlockSpec((1, DM), lambda i: (0, 0)),
            pl.BlockSpec((DM, DH), lambda i: (0, 0)),
            pl.BlockSpec((1, DH), lambda i: (0, 0)),
            pl.BlockSpec((DH, DM), lambda i: (0, 0)),
            pl.BlockSpec((1, DM), lambda i: (0, 0)),
            pl.BlockSpec((1, DM), lambda i: (0, 0)),
            pl.BlockSpec((1, 1), lambda i: (0, 0)),
        ],
        out_specs=[
            pl.BlockSpec((R, DM), lambda i: (i, 0)),
            pl.BlockSpec((1, 1, R), lambda i: (i, 0, 0)),
        ],
        out_shape=[
            jax.ShapeDtypeStruct((T * D, DM), bf16),
            jax.ShapeDtypeStruct((n_a, 1, R), f32),
        ],
    )(
        H2, E, depth_emb,
        ln_g.reshape(1, DM), ln_b.reshape(1, DM),
        W1.astype(bf16), b1.reshape(1, DH),
        W2.astype(bf16), b2.reshape(1, DM),
        Wb.reshape(1, DM), bb.reshape(1, 1),
    )

    n_b = (T * D) // _RB
    q2 = pl.pallas_call(
        _q_body,
        grid=(n_b,),
        in_specs=[
            pl.BlockSpec((_RB, DM), lambda i: (i, 0)),
            pl.BlockSpec((DM, V), lambda i: (0, 0)),
            pl.BlockSpec((1, V), lambda i: (0, 0)),
        ],
        out_specs=pl.BlockSpec((_RB, V), lambda i: (i, 0)),
        out_shape=jax.ShapeDtypeStruct((T * D, V), f32),
    )(s2_flat, Wq.astype(bf16), bq.reshape(1, V))

    q = q2.reshape(B, T, D, V)
    beta = beta2.reshape(B, T, D)
    return (q, beta)


# speedup vs baseline: 2.3167x; 1.6144x over previous
import functools
import math

import jax
import jax.numpy as jnp
from jax.experimental import pallas as pl
from jax.experimental.pallas import tpu as pltpu
from jax.experimental.pallas import tpu_sc as plsc

B, T, D, V, DM, DH = 1, 2048, 4, 8192, 1024, 4096
LN_EPS = 1e-05

_GATHER_W = 128
_GK = 4
_GC = DM // _GK
_TT = 64
_QT = 32


def _sc_gather(emb4, ids_exp):
    mesh = plsc.VectorSubcoreMesh(core_axis_name="core", subcore_axis_name="subcore")
    n_idx = T * _GK

    @functools.partial(
        pl.kernel,
        out_type=jax.ShapeDtypeStruct((n_idx, _GC), emb4.dtype),
        mesh=mesh,
    )
    def gather_kernel(emb_hbm, ids_hbm, out_hbm):
        def body(i_vmem, o_vmem):
            pltpu.sync_copy(emb_hbm.at[i_vmem.at[0]], o_vmem)

        pltpu.emit_pipeline(
            body,
            grid=(n_idx // _GATHER_W,),
            in_specs=[pl.BlockSpec((1, _GATHER_W), index_map=lambda i: (0, i))],
            out_specs=[pl.BlockSpec((_GATHER_W, _GC), index_map=lambda i: (i, 0))],
            core_axis_name=("core", "subcore"),
            dimension_semantics=(pltpu.PARALLEL,),
        )(ids_hbm, out_hbm)

    return gather_kernel(emb4, ids_exp)


def _mlp_body(h_ref, e_ref, demb_ref, lng_ref, lnb_ref, w1_ref, b1_ref,
              w2_ref, b2_ref, wb_ref, bb_ref, s2_ref, beta_ref):
    R = _TT * D
    e = e_ref[...].reshape(_TT, DM)
    x = h_ref[...] + e
    x4 = jnp.broadcast_to(x[:, None, :], (_TT, D, DM)).reshape(R, DM)
    d4 = jnp.broadcast_to(demb_ref[...][None, :, :], (_TT, D, DM)).reshape(R, DM)
    s = x4 + d4
    mu = jnp.mean(s, axis=1, keepdims=True)
    c = s - mu
    var = jnp.mean(c * c, axis=1, keepdims=True)
    sn = c * jax.lax.rsqrt(var + LN_EPS) * lng_ref[...] + lnb_ref[...]
    hpre = jnp.dot(sn.astype(jnp.bfloat16), w1_ref[...],
                   preferred_element_type=jnp.float32) + b1_ref[...]
    h = 0.5 * hpre * (1.0 + jax.lax.erf(hpre * (1.0 / math.sqrt(2.0))))
    s2 = jnp.dot(h.astype(jnp.bfloat16), w2_ref[...],
                 preferred_element_type=jnp.float32) + b2_ref[...]
    s2_ref[...] = s2.astype(jnp.bfloat16)
    beta_ref[...] = (jnp.sum(s2 * wb_ref[...], axis=1) + bb_ref[0, 0]).reshape(1, _TT, D)


def _q_body(s2_ref, wq_ref, bq_ref, q_ref):
    qt = jnp.dot(s2_ref[...], wq_ref[...],
                 preferred_element_type=jnp.float32) + bq_ref[...]
    q_ref[...] = qt.reshape(1, _QT, D, V)


def kernel(H, input_ids, emb, depth_emb, ln_g, ln_b, W1, b1, W2, b2, Wq, bq, Wb, bb):
    f32 = jnp.float32
    bf16 = jnp.bfloat16

    H2 = H.reshape(T, DM)
    ids = input_ids.reshape(T).astype(jnp.int32)
    ids_exp = (ids[:, None] * _GK + jnp.arange(_GK, dtype=jnp.int32)[None, :]
               ).reshape(1, T * _GK)

    E = _sc_gather(emb.reshape(V * _GK, _GC), ids_exp)

    R = _TT * D
    n_a = T // _TT
    s2_flat, beta = pl.pallas_call(
        _mlp_body,
        grid=(n_a,),
        in_specs=[
            pl.BlockSpec((_TT, DM), lambda i: (i, 0)),
            pl.BlockSpec((_TT * _GK, _GC), lambda i: (i, 0)),
            pl.BlockSpec((D, DM), lambda i: (0, 0)),
            pl.BlockSpec((1, DM), lambda i: (0, 0)),
            pl.BlockSpec((1, DM), lambda i: (0, 0)),
            pl.BlockSpec((DM, DH), lambda i: (0, 0)),
            pl.BlockSpec((1, DH), lambda i: (0, 0)),
            pl.BlockSpec((DH, DM), lambda i: (0, 0)),
            pl.BlockSpec((1, DM), lambda i: (0, 0)),
            pl.BlockSpec((1, DM), lambda i: (0, 0)),
            pl.BlockSpec((1, 1), lambda i: (0, 0)),
        ],
        out_specs=[
            pl.BlockSpec((R, DM), lambda i: (i, 0)),
            pl.BlockSpec((1, _TT, D), lambda i: (0, i, 0)),
        ],
        out_shape=[
            jax.ShapeDtypeStruct((T * D, DM), bf16),
            jax.ShapeDtypeStruct((B, T, D), f32),
        ],
    )(
        H2, E, depth_emb,
        ln_g.reshape(1, DM), ln_b.reshape(1, DM),
        W1.astype(bf16), b1.reshape(1, DH),
        W2.astype(bf16), b2.reshape(1, DM),
        Wb.reshape(1, DM), bb.reshape(1, 1),
    )

    n_b = T // _QT
    q = pl.pallas_call(
        _q_body,
        grid=(n_b,),
        in_specs=[
            pl.BlockSpec((_QT * D, DM), lambda i: (i, 0)),
            pl.BlockSpec((DM, V), lambda i: (0, 0)),
            pl.BlockSpec((1, V), lambda i: (0, 0)),
        ],
        out_specs=pl.BlockSpec((1, _QT, D, V), lambda i: (0, i, 0, 0)),
        out_shape=jax.ShapeDtypeStruct((B, T, D, V), f32),
    )(s2_flat, Wq.astype(bf16), bq.reshape(1, V))

    return (q, beta)


# speedup vs baseline: 2.5600x; 1.1050x over previous
import functools
import math

import jax
import jax.numpy as jnp
from jax.experimental import pallas as pl
from jax.experimental.pallas import tpu as pltpu
from jax.experimental.pallas import tpu_sc as plsc

B, T, D, V, DM, DH = 1, 2048, 4, 8192, 1024, 4096
LN_EPS = 1e-05

_NSC = 32
_TT = 64
_QT = 32


def _sc_gather(emb, ids):
    mesh = plsc.VectorSubcoreMesh(core_axis_name="c", subcore_axis_name="s")
    per_w = T // _NSC

    @functools.partial(
        pl.kernel,
        out_type=jax.ShapeDtypeStruct((T, DM), emb.dtype),
        mesh=mesh,
        scratch_types=[
            pltpu.VMEM((per_w,), jnp.int32),
            pltpu.VMEM((per_w, DM), jnp.float32),
            pltpu.SemaphoreType.DMA,
        ],
    )
    def gather_kernel(emb_hbm, ids_hbm, out_hbm, idx_v, rows_v, sem):
        wid = jax.lax.axis_index("s") * 2 + jax.lax.axis_index("c")
        base = wid * per_w
        pltpu.sync_copy(ids_hbm.at[pl.ds(base, per_w)], idx_v)
        pltpu.async_copy(emb_hbm.at[idx_v], rows_v, sem).wait()
        pltpu.sync_copy(rows_v, out_hbm.at[pl.ds(base, per_w)])

    return gather_kernel(emb, ids)


def _mlp_body(h_ref, e_ref, demb_ref, lng_ref, lnb_ref, w1_ref, b1_ref,
              w2_ref, b2_ref, wb_ref, bb_ref, s2_ref, beta_ref):
    R = _TT * D
    x = h_ref[...] + e_ref[...]
    x4 = jnp.broadcast_to(x[:, None, :], (_TT, D, DM)).reshape(R, DM)
    d4 = jnp.broadcast_to(demb_ref[...][None, :, :], (_TT, D, DM)).reshape(R, DM)
    s = x4 + d4
    mu = jnp.mean(s, axis=1, keepdims=True)
    c = s - mu
    var = jnp.mean(c * c, axis=1, keepdims=True)
    sn = c * jax.lax.rsqrt(var + LN_EPS) * lng_ref[...] + lnb_ref[...]
    hpre = jnp.dot(sn.astype(jnp.bfloat16), w1_ref[...],
                   preferred_element_type=jnp.float32) + b1_ref[...]
    h = 0.5 * hpre * (1.0 + jax.lax.erf(hpre * (1.0 / math.sqrt(2.0))))
    s2 = jnp.dot(h.astype(jnp.bfloat16), w2_ref[...],
                 preferred_element_type=jnp.float32) + b2_ref[...]
    s2_ref[...] = s2.astype(jnp.bfloat16)
    beta_ref[...] = (jnp.sum(s2 * wb_ref[...], axis=1) + bb_ref[0, 0]).reshape(1, _TT, D)


def _q_body(s2_ref, wq_ref, bq_ref, q_ref):
    qt = jnp.dot(s2_ref[...], wq_ref[...],
                 preferred_element_type=jnp.float32) + bq_ref[...]
    q_ref[...] = qt.reshape(1, _QT, D, V)


def kernel(H, input_ids, emb, depth_emb, ln_g, ln_b, W1, b1, W2, b2, Wq, bq, Wb, bb):
    f32 = jnp.float32
    bf16 = jnp.bfloat16

    H2 = H.reshape(T, DM)
    ids = input_ids.reshape(T).astype(jnp.int32)

    E = _sc_gather(emb, ids)

    R = _TT * D
    n_a = T // _TT
    s2_flat, beta = pl.pallas_call(
        _mlp_body,
        grid=(n_a,),
        in_specs=[
            pl.BlockSpec((_TT, DM), lambda i: (i, 0)),
            pl.BlockSpec((_TT, DM), lambda i: (i, 0)),
            pl.BlockSpec((D, DM), lambda i: (0, 0)),
            pl.BlockSpec((1, DM), lambda i: (0, 0)),
            pl.BlockSpec((1, DM), lambda i: (0, 0)),
            pl.BlockSpec((DM, DH), lambda i: (0, 0)),
            pl.BlockSpec((1, DH), lambda i: (0, 0)),
            pl.BlockSpec((DH, DM), lambda i: (0, 0)),
            pl.BlockSpec((1, DM), lambda i: (0, 0)),
            pl.BlockSpec((1, DM), lambda i: (0, 0)),
            pl.BlockSpec((1, 1), lambda i: (0, 0)),
        ],
        out_specs=[
            pl.BlockSpec((R, DM), lambda i: (i, 0)),
            pl.BlockSpec((1, _TT, D), lambda i: (0, i, 0)),
        ],
        out_shape=[
            jax.ShapeDtypeStruct((T * D, DM), bf16),
            jax.ShapeDtypeStruct((B, T, D), f32),
        ],
    )(
        H2, E, depth_emb,
        ln_g.reshape(1, DM), ln_b.reshape(1, DM),
        W1.astype(bf16), b1.reshape(1, DH),
        W2.astype(bf16), b2.reshape(1, DM),
        Wb.reshape(1, DM), bb.reshape(1, 1),
    )

    n_b = T // _QT
    q = pl.pallas_call(
        _q_body,
        grid=(n_b,),
        in_specs=[
            pl.BlockSpec((_QT * D, DM), lambda i: (i, 0)),
            pl.BlockSpec((DM, V), lambda i: (0, 0)),
            pl.BlockSpec((1, V), lambda i: (0, 0)),
        ],
        out_specs=pl.BlockSpec((1, _QT, D, V), lambda i: (0, i, 0, 0)),
        out_shape=jax.ShapeDtypeStruct((B, T, D, V), f32),
    )(s2_flat, Wq.astype(bf16), bq.reshape(1, V))

    return (q, beta)


# speedup vs baseline: 2.6702x; 1.0431x over previous
import functools
import math

import jax
import jax.numpy as jnp
from jax.experimental import pallas as pl
from jax.experimental.pallas import tpu as pltpu
from jax.experimental.pallas import tpu_sc as plsc

B, T, D, V, DM, DH = 1, 2048, 4, 8192, 1024, 4096
LN_EPS = 1e-05

_NSC = 32
_TT = 128
_QT = 64


def _sc_gather(emb, ids):
    mesh = plsc.VectorSubcoreMesh(core_axis_name="c", subcore_axis_name="s")
    per_w = T // _NSC

    @functools.partial(
        pl.kernel,
        out_type=jax.ShapeDtypeStruct((T, DM), emb.dtype),
        mesh=mesh,
        scratch_types=[
            pltpu.VMEM((per_w,), jnp.int32),
            pltpu.VMEM((per_w, DM), jnp.float32),
            pltpu.SemaphoreType.DMA,
        ],
    )
    def gather_kernel(emb_hbm, ids_hbm, out_hbm, idx_v, rows_v, sem):
        wid = jax.lax.axis_index("s") * 2 + jax.lax.axis_index("c")
        base = wid * per_w
        pltpu.sync_copy(ids_hbm.at[pl.ds(base, per_w)], idx_v)
        pltpu.async_copy(emb_hbm.at[idx_v], rows_v, sem).wait()
        pltpu.sync_copy(rows_v, out_hbm.at[pl.ds(base, per_w)])

    return gather_kernel(emb, ids)


def _mlp_body(h_ref, e_ref, demb_ref, lng_ref, lnb_ref, w1_ref, b1_ref,
              w2_ref, b2_ref, wb_ref, bb_ref, s2_ref, beta_ref):
    R = _TT * D
    x = h_ref[...] + e_ref[...]
    x4 = jnp.broadcast_to(x[:, None, :], (_TT, D, DM)).reshape(R, DM)
    d4 = jnp.broadcast_to(demb_ref[...][None, :, :], (_TT, D, DM)).reshape(R, DM)
    s = x4 + d4
    mu = jnp.mean(s, axis=1, keepdims=True)
    c = s - mu
    var = jnp.mean(c * c, axis=1, keepdims=True)
    sn = c * jax.lax.rsqrt(var + LN_EPS) * lng_ref[...] + lnb_ref[...]
    hpre = jnp.dot(sn.astype(jnp.bfloat16), w1_ref[...],
                   preferred_element_type=jnp.float32) + b1_ref[...]
    h = 0.5 * hpre * (1.0 + jax.lax.erf(hpre * (1.0 / math.sqrt(2.0))))
    s2 = jnp.dot(h.astype(jnp.bfloat16), w2_ref[...],
                 preferred_element_type=jnp.float32) + b2_ref[...]
    s2_ref[...] = s2.astype(jnp.bfloat16)
    beta_ref[...] = (jnp.sum(s2 * wb_ref[...], axis=1) + bb_ref[0, 0]).reshape(1, _TT, D)


def _q_body(s2_ref, wq_ref, bq_ref, q_ref):
    qt = jnp.dot(s2_ref[...], wq_ref[...],
                 preferred_element_type=jnp.float32) + bq_ref[...]
    q_ref[...] = qt.reshape(1, _QT, D, V)


def kernel(H, input_ids, emb, depth_emb, ln_g, ln_b, W1, b1, W2, b2, Wq, bq, Wb, bb):
    f32 = jnp.float32
    bf16 = jnp.bfloat16

    H2 = H.reshape(T, DM)
    ids = input_ids.reshape(T).astype(jnp.int32)

    E = _sc_gather(emb, ids)

    R = _TT * D
    n_a = T // _TT
    s2_flat, beta = pl.pallas_call(
        _mlp_body,
        grid=(n_a,),
        in_specs=[
            pl.BlockSpec((_TT, DM), lambda i: (i, 0)),
            pl.BlockSpec((_TT, DM), lambda i: (i, 0)),
            pl.BlockSpec((D, DM), lambda i: (0, 0)),
            pl.BlockSpec((1, DM), lambda i: (0, 0)),
            pl.BlockSpec((1, DM), lambda i: (0, 0)),
            pl.BlockSpec((DM, DH), lambda i: (0, 0)),
            pl.BlockSpec((1, DH), lambda i: (0, 0)),
            pl.BlockSpec((DH, DM), lambda i: (0, 0)),
            pl.BlockSpec((1, DM), lambda i: (0, 0)),
            pl.BlockSpec((1, DM), lambda i: (0, 0)),
            pl.BlockSpec((1, 1), lambda i: (0, 0)),
        ],
        out_specs=[
            pl.BlockSpec((R, DM), lambda i: (i, 0)),
            pl.BlockSpec((1, _TT, D), lambda i: (0, i, 0)),
        ],
        out_shape=[
            jax.ShapeDtypeStruct((T * D, DM), bf16),
            jax.ShapeDtypeStruct((B, T, D), f32),
        ],
    )(
        H2, E, depth_emb,
        ln_g.reshape(1, DM), ln_b.reshape(1, DM),
        W1.astype(bf16), b1.reshape(1, DH),
        W2.astype(bf16), b2.reshape(1, DM),
        Wb.reshape(1, DM), bb.reshape(1, 1),
    )

    n_b = T // _QT
    q = pl.pallas_call(
        _q_body,
        grid=(n_b,),
        in_specs=[
            pl.BlockSpec((_QT * D, DM), lambda i: (i, 0)),
            pl.BlockSpec((DM, V), lambda i: (0, 0)),
            pl.BlockSpec((1, V), lambda i: (0, 0)),
        ],
        out_specs=pl.BlockSpec((1, _QT, D, V), lambda i: (0, i, 0, 0)),
        out_shape=jax.ShapeDtypeStruct((B, T, D, V), f32),
    )(s2_flat, Wq.astype(bf16), bq.reshape(1, V))

    return (q, beta)


# speedup vs baseline: 2.6809x; 1.0040x over previous
import functools
import math

import jax
import jax.numpy as jnp
from jax.experimental import pallas as pl
from jax.experimental.pallas import tpu as pltpu
from jax.experimental.pallas import tpu_sc as plsc

B, T, D, V, DM, DH = 1, 2048, 4, 8192, 1024, 4096
LN_EPS = 1e-05

_NSC = 32
_TT = 128
_QR = 1024
_QC = 2048


def _sc_gather(emb, ids):
    mesh = plsc.VectorSubcoreMesh(core_axis_name="c", subcore_axis_name="s")
    per_w = T // _NSC

    @functools.partial(
        pl.kernel,
        out_type=jax.ShapeDtypeStruct((T, DM), emb.dtype),
        mesh=mesh,
        scratch_types=[
            pltpu.VMEM((per_w,), jnp.int32),
            pltpu.VMEM((per_w, DM), jnp.float32),
            pltpu.SemaphoreType.DMA,
        ],
    )
    def gather_kernel(emb_hbm, ids_hbm, out_hbm, idx_v, rows_v, sem):
        wid = jax.lax.axis_index("s") * 2 + jax.lax.axis_index("c")
        base = wid * per_w
        pltpu.sync_copy(ids_hbm.at[pl.ds(base, per_w)], idx_v)
        pltpu.async_copy(emb_hbm.at[idx_v], rows_v, sem).wait()
        pltpu.sync_copy(rows_v, out_hbm.at[pl.ds(base, per_w)])

    return gather_kernel(emb, ids)


def _mlp_body(h_ref, e_ref, demb_ref, lng_ref, lnb_ref, w1_ref, b1_ref,
              w2_ref, b2_ref, wb_ref, bb_ref, s2_ref, beta_ref):
    R = _TT * D
    x = h_ref[...] + e_ref[...]
    x4 = jnp.broadcast_to(x[:, None, :], (_TT, D, DM)).reshape(R, DM)
    d4 = jnp.broadcast_to(demb_ref[...][None, :, :], (_TT, D, DM)).reshape(R, DM)
    s = x4 + d4
    mu = jnp.mean(s, axis=1, keepdims=True)
    c = s - mu
    var = jnp.mean(c * c, axis=1, keepdims=True)
    sn = c * jax.lax.rsqrt(var + LN_EPS) * lng_ref[...] + lnb_ref[...]
    hpre = jnp.dot(sn.astype(jnp.bfloat16), w1_ref[...],
                   preferred_element_type=jnp.float32) + b1_ref[...]
    h = 0.5 * hpre * (1.0 + jax.lax.erf(hpre * (1.0 / math.sqrt(2.0))))
    s2 = jnp.dot(h.astype(jnp.bfloat16), w2_ref[...],
                 preferred_element_type=jnp.float32) + b2_ref[...]
    s2_ref[...] = s2.astype(jnp.bfloat16)
    beta_ref[...] = (jnp.sum(s2 * wb_ref[...], axis=1) + bb_ref[0, 0]).reshape(1, _TT, D)


def _q_body(s2_ref, wq_ref, bq_ref, q_ref):
    qt = jnp.dot(s2_ref[...], wq_ref[...],
                 preferred_element_type=jnp.float32) + bq_ref[...]
    q_ref[...] = qt.reshape(1, _QR // D, D, _QC)


def kernel(H, input_ids, emb, depth_emb, ln_g, ln_b, W1, b1, W2, b2, Wq, bq, Wb, bb):
    f32 = jnp.float32
    bf16 = jnp.bfloat16

    H2 = H.reshape(T, DM)
    ids = input_ids.reshape(T).astype(jnp.int32)

    E = _sc_gather(emb, ids)

    R = _TT * D
    n_a = T // _TT
    s2_flat, beta = pl.pallas_call(
        _mlp_body,
        grid=(n_a,),
        in_specs=[
            pl.BlockSpec((_TT, DM), lambda i: (i, 0)),
            pl.BlockSpec((_TT, DM), lambda i: (i, 0)),
            pl.BlockSpec((D, DM), lambda i: (0, 0)),
            pl.BlockSpec((1, DM), lambda i: (0, 0)),
            pl.BlockSpec((1, DM), lambda i: (0, 0)),
            pl.BlockSpec((DM, DH), lambda i: (0, 0)),
            pl.BlockSpec((1, DH), lambda i: (0, 0)),
            pl.BlockSpec((DH, DM), lambda i: (0, 0)),
            pl.BlockSpec((1, DM), lambda i: (0, 0)),
            pl.BlockSpec((1, DM), lambda i: (0, 0)),
            pl.BlockSpec((1, 1), lambda i: (0, 0)),
        ],
        out_specs=[
            pl.BlockSpec((R, DM), lambda i: (i, 0)),
            pl.BlockSpec((1, _TT, D), lambda i: (0, i, 0)),
        ],
        out_shape=[
            jax.ShapeDtypeStruct((T * D, DM), bf16),
            jax.ShapeDtypeStruct((B, T, D), f32),
        ],
    )(
        H2, E, depth_emb,
        ln_g.reshape(1, DM), ln_b.reshape(1, DM),
        W1.astype(bf16), b1.reshape(1, DH),
        W2.astype(bf16), b2.reshape(1, DM),
        Wb.reshape(1, DM), bb.reshape(1, 1),
    )

    n_qr = (T * D) // _QR
    n_qc = V // _QC
    q = pl.pallas_call(
        _q_body,
        grid=(n_qc, n_qr),
        in_specs=[
            pl.BlockSpec((_QR, DM), lambda j, i: (i, 0)),
            pl.BlockSpec((DM, _QC), lambda j, i: (0, j)),
            pl.BlockSpec((1, _QC), lambda j, i: (0, j)),
        ],
        out_specs=pl.BlockSpec((1, _QR // D, D, _QC),
                               lambda j, i: (0, i, 0, j)),
        out_shape=jax.ShapeDtypeStruct((B, T, D, V), f32),
    )(s2_flat, Wq.astype(bf16), bq.reshape(1, V))

    return (q, beta)
